# Initial kernel scaffold; baseline (speedup 1.0000x reference)
import jax
import jax.numpy as jnp
from jax.experimental import pallas as pl


def kernel(radical_indices, position_labels, rademb_table, posemb_table):
    raise NotImplementedError("write your pallas kernel here")


# speedup vs baseline: 4.0570x; 4.0570x over previous
import functools

import jax
import jax.numpy as jnp
from jax import lax
from jax.experimental import pallas as pl
from jax.experimental.pallas import tpu as pltpu
from jax.experimental.pallas import tpu_sc as plsc

B = 16384
L = 20
N = B * L
RAD_D = 32
POS_D = 128
OUT_D = RAD_D + POS_D

NC = 2
NS = 16
NW = NC * NS
N_PER_W = N // NW
G = 128
C = 512
N_ITERS = N_PER_W // C


def _renorm_rows(x):
    ss = jnp.sum(x * x, axis=-1, keepdims=True)
    norm = jnp.sqrt(ss)
    scale = jnp.minimum(1.0, 1.0 / jnp.maximum(norm, 1e-7))
    return x * scale


def _renorm_kernel(tab_ref, out_ref):
    out_ref[...] = _renorm_rows(tab_ref[...])


def _prescale_rad(table):
    blk = 2048
    return pl.pallas_call(
        _renorm_kernel,
        grid=(pl.cdiv(table.shape[0], blk),),
        in_specs=[pl.BlockSpec((blk, RAD_D), lambda i: (i, 0))],
        out_specs=pl.BlockSpec((blk, RAD_D), lambda i: (i, 0)),
        out_shape=jax.ShapeDtypeStruct(table.shape, table.dtype),
    )(table)


def _prescale_pos(table):
    return pl.pallas_call(
        _renorm_kernel,
        out_shape=jax.ShapeDtypeStruct(table.shape, table.dtype),
    )(table)


def _sc_gather_concat(rad_tab, pos_tab, rad_idx2d, pos_idx2d):
    mesh = plsc.VectorSubcoreMesh(core_axis_name="c", subcore_axis_name="s")

    @functools.partial(
        pl.kernel,
        out_type=jax.ShapeDtypeStruct((N, OUT_D), jnp.float32),
        mesh=mesh,
        compiler_params=pltpu.CompilerParams(use_tc_tiling_on_sc=False),
        scratch_types=[
            pltpu.VMEM((C // G, G), jnp.int32),
            pltpu.VMEM((C // G, G), jnp.int32),
            pltpu.VMEM((C, RAD_D), jnp.float32),
            pltpu.VMEM((C, POS_D), jnp.float32),
            pltpu.SemaphoreType.DMA,
        ],
    )
    def k(rad_tab_hbm, pos_tab_hbm, ridx_hbm, pidx_hbm, out_hbm,
          ridx_v, pidx_v, rrows_v, prows_v, sem):
        wid = lax.axis_index("s") * NC + lax.axis_index("c")
        row0 = wid * (N_PER_W // G)

        def body(j, carry):
            irow = row0 + j * (C // G)
            base = irow * G
            pltpu.sync_copy(ridx_hbm.at[pl.ds(irow, C // G)], ridx_v)
            pltpu.sync_copy(pidx_hbm.at[pl.ds(irow, C // G)], pidx_v)
            copies = []
            for kk in range(C // G):
                copies.append(pltpu.async_copy(
                    rad_tab_hbm.at[ridx_v.at[kk]],
                    rrows_v.at[pl.ds(kk * G, G)], sem))
                copies.append(pltpu.async_copy(
                    pos_tab_hbm.at[pidx_v.at[kk]],
                    prows_v.at[pl.ds(kk * G, G)], sem))
            for cp in copies:
                cp.wait()
            pltpu.sync_copy(rrows_v, out_hbm.at[pl.ds(base, C), pl.ds(0, RAD_D)])
            pltpu.sync_copy(prows_v, out_hbm.at[pl.ds(base, C), pl.ds(RAD_D, POS_D)])
            return carry

        lax.fori_loop(0, N_ITERS, body, 0, unroll=False)

    return k(rad_tab, pos_tab, rad_idx2d, pos_idx2d)


def kernel(radical_indices, position_labels, rademb_table, posemb_table):
    rad_tab = _prescale_rad(rademb_table)
    pos_tab = _prescale_pos(posemb_table)
    ridx = radical_indices.reshape(N // G, G).astype(jnp.int32)
    pidx = position_labels.reshape(N // G, G).astype(jnp.int32)
    out = _sc_gather_concat(rad_tab, pos_tab, ridx, pidx)
    return out.reshape(B, L, OUT_D)
